# baseline (speedup 1.0000x reference)
import functools

import jax
import jax.numpy as jnp
from jax import lax
from jax.experimental import pallas as pl
from jax.experimental.pallas import tpu as pltpu
from jax.experimental.pallas import tpu_sc as plsc

NC = 2
NS = 16
NW = NC * NS

G = 128
L = 16


def _gather_t_sc(table, idxT, n_items, d):
    per_w = n_items // NW
    nt = per_w // 2
    nh = n_items // G

    mesh = plsc.VectorSubcoreMesh(core_axis_name="c", subcore_axis_name="s")

    @functools.partial(
        pl.kernel,
        out_type=jax.ShapeDtypeStruct((nh, d // 8, G, 8, G), jnp.float32),
        mesh=mesh,
        scratch_types=[
            pltpu.VMEM((per_w, G), jnp.int32),
            pltpu.VMEM((G, d), jnp.float32),
            pltpu.VMEM((G, d), jnp.float32),
            pltpu.VMEM((d // 8, 8, G), jnp.float32),
            pltpu.VMEM((d // 8, 8, G), jnp.float32),
            pltpu.SemaphoreType.DMA,
            pltpu.SemaphoreType.DMA,
        ],
        compiler_params=pltpu.CompilerParams(
            use_tc_tiling_on_sc=False, needs_layout_passes=False,
            disable_bounds_checks=True),
    )
    def k(tab_hbm, idx_hbm, o5_hbm, idxv, g0, g1, t0, t1, sem_g, sem_o):
        wid = lax.axis_index("s") * NC + lax.axis_index("c")
        base = wid * per_w

        pltpu.sync_copy(idx_hbm.at[pl.ds(base, per_w)], idxv)

        riota = lax.iota(jnp.int32, L)
        rowidx = [riota + (L * v) for v in range(G // L)]

        def transpose_block(g, t):
            @plsc.parallel_loop(0, d, 1, unroll=4)
            def _(c):
                cvec = jnp.broadcast_to(c, (L,))
                c2 = lax.shift_right_logical(c, 3)
                c1 = lax.bitwise_and(c, 7)
                for v in range(G // L):
                    vals = plsc.load_gather(g, [rowidx[v], cvec])
                    t[c2, c1, pl.ds(L * v, L)] = vals

        def out_dst(item):
            h = lax.shift_right_logical(item, 7)
            b128 = lax.bitwise_and(item, 127)
            return o5_hbm.at[h, :, b128]

        pltpu.async_copy(tab_hbm.at[idxv.at[0]], g0, sem_g)

        def body(t_i, carry):
            i0 = 2 * t_i
            for u, (g, tt, go) in enumerate(((g0, t0, g1), (g1, t1, g0))):
                i = i0 + u
                pltpu.make_async_copy(tab_hbm.at[idxv.at[i]], g, sem_g).wait()
                if u == 0:
                    pltpu.async_copy(tab_hbm.at[idxv.at[i + 1]], go, sem_g)
                else:
                    @pl.when(t_i < nt - 1)
                    def _():
                        pltpu.async_copy(
                            tab_hbm.at[idxv.at[i + 1]], go, sem_g)
                @pl.when(t_i > 0)
                def _():
                    pltpu.make_async_copy(
                        tt, out_dst(base + i - 2), sem_o).wait()
                transpose_block(g, tt)
                pltpu.async_copy(tt, out_dst(base + i), sem_o)
            return carry

        lax.fori_loop(0, nt, body, 0)

        pltpu.make_async_copy(t0, out_dst(base + per_w - 2), sem_o).wait()
        pltpu.make_async_copy(t1, out_dst(base + per_w - 1), sem_o).wait()

    return k(table, idxT)


def kernel(message, weight):
    b, h = message.shape
    v, d = weight.shape
    n = b * h
    idxT = message.T.reshape(n // G, G)
    o5 = _gather_t_sc(weight, idxT, n // G, d)
    out = o5.transpose(2, 4, 0, 1, 3).reshape(b, h, d)
    return out
